# baseline (speedup 1.0000x reference)
import jax
import jax.numpy as jnp
from jax.experimental import pallas as pl
from jax.experimental.pallas import tpu as pltpu
from jax.experimental.pallas import tpu_sc as plsc


_NB = 128


def _sc_gather_items(table, idx, n_l, n_b, window):
    d = table.shape[1]
    n = idx.shape[0]
    per_l = n_b // window
    mesh = plsc.VectorSubcoreMesh(core_axis_name="core", subcore_axis_name="subcore")

    @pl.kernel(out_type=jax.ShapeDtypeStruct((n_l, n_b, d), table.dtype),
               mesh=mesh)
    def kern(tab_hbm, i_hbm, o_hbm):
        def body(i_vmem, o_vmem):
            pltpu.sync_copy(tab_hbm.at[i_vmem.at[0]], o_vmem.at[0])

        pltpu.emit_pipeline(
            body,
            grid=(n // window,),
            in_specs=[pl.BlockSpec((1, window), lambda i: (0, i))],
            out_specs=[pl.BlockSpec((1, window, d),
                                    lambda i: (i // per_l, i % per_l, 0))],
            core_axis_name=("core", "subcore"),
            dimension_semantics=(pltpu.PARALLEL,),
        )(i_hbm, o_hbm)

    return kern(table, idx.reshape(1, n))


def _tc_body(gp_ref, parf_ref, ratf_ref, nodes_ref, utab_ref, rtab_ref,
             wiA_ref, wiB_ref, wr_ref, ln1b_ref, wa_ref, wau_ref, att1b_ref,
             att2w_ref, att2b_ref, att3w_ref, ln2w_ref, ln2b_ref,
             w3u_ref, w3i_ref, ln3b_ref, out_ref, u_scr, u_sem):
    f32 = jnp.float32

    def dot_t(x, w):
        return jax.lax.dot_general(x, w, (((1,), (1,)), ((), ())),
                                   preferred_element_type=f32)

    L, nB, D2 = gp_ref.shape
    D = D2 // 2

    def issue(j, _):
        r = nodes_ref[0, 0, j]
        pltpu.make_async_copy(utab_ref.at[pl.ds(r, 1), :],
                              u_scr.at[pl.ds(j, 1), :], u_sem).start()
        return _
    jax.lax.fori_loop(0, nB, issue, None)

    gp2 = gp_ref[...].reshape(L * nB, D2)
    even_part = dot_t(gp2, wiA_ref[...]).reshape(L, nB, D)
    odd_part = dot_t(gp2, wiB_ref[...]).reshape(L, nB, D)
    item_part = jnp.where(parf_ref[...] < 0.5, even_part, odd_part)

    pre_rat = dot_t(rtab_ref[...], wr_ref[...]) + ln1b_ref[...]
    p = [pre_rat[k:k + 1][None] for k in range(5)]
    r3 = ratf_ref[...]
    v01 = jnp.where(r3 < 0.5, p[0], p[1])
    v23 = jnp.where(r3 < 2.5, p[2], p[3])
    v0123 = jnp.where(r3 < 1.5, v01, v23)
    ratc = jnp.where(r3 < 3.5, v0123, p[4])
    xr3 = jnp.maximum(item_part + ratc, 0.0)
    xr2 = xr3.reshape(L * nB, D)

    def drain(j, _):
        pltpu.make_async_copy(utab_ref.at[pl.ds(0, 1), :],
                              u_scr.at[pl.ds(0, 1), :], u_sem).wait()
        return _
    jax.lax.fori_loop(0, nB, drain, None)
    u = u_scr[...]

    u_att = dot_t(u, wau_ref[...]) + att1b_ref[...]
    a1 = dot_t(xr2, wa_ref[...]).reshape(L, nB, D) + u_att[None]
    a1 = jnp.maximum(a1, 0.0)
    a2 = jnp.maximum(dot_t(a1.reshape(L * nB, D), att2w_ref[...])
                     + att2b_ref[...], 0.0)

    s = jnp.sum(a2.reshape(L, nB, D) * att3w_ref[...][None], axis=2,
                keepdims=True)
    m = jnp.max(s, axis=0, keepdims=True)
    e = jnp.exp(s - m)
    denom = jnp.sum(e, axis=0, keepdims=True)
    hI = jnp.sum(xr3 * (e / denom), axis=0)

    h2 = jnp.maximum(dot_t(hI, ln2w_ref[...]) + ln2b_ref[...], 0.0)
    out = dot_t(u, w3u_ref[...]) + dot_t(h2, w3i_ref[...]) + ln3b_ref[...]
    out_ref[...] = jnp.maximum(out, 0.0)


def _tc_compute(gp3, parf, ratf, nodes_b, user_table, rating_table,
                weights):
    L, B, D2 = gp3.shape
    D = D2 // 2
    grid = (B // _NB,)
    full = lambda a: pl.BlockSpec(a.shape, lambda i: tuple(0 for _ in a.shape))
    return pl.pallas_call(
        _tc_body,
        grid=grid,
        in_specs=[
            pl.BlockSpec((L, _NB, D2), lambda i: (0, i, 0)),
            pl.BlockSpec((L, _NB, 1), lambda i: (0, i, 0)),
            pl.BlockSpec((L, _NB, 1), lambda i: (0, i, 0)),
            pl.BlockSpec((1, 1, _NB), lambda i: (i, 0, 0),
                         memory_space=pltpu.SMEM),
            pl.BlockSpec(memory_space=pl.ANY),
            full(rating_table),
        ] + [full(w) for w in weights],
        out_specs=pl.BlockSpec((_NB, D), lambda i: (i, 0)),
        out_shape=jax.ShapeDtypeStruct((B, D), jnp.float32),
        scratch_shapes=[pltpu.VMEM((_NB, D), jnp.float32),
                        pltpu.SemaphoreType.DMA],
        compiler_params=pltpu.CompilerParams(
            dimension_semantics=("parallel",)),
    )(gp3, parf, ratf, nodes_b, user_table, rating_table, *weights)


def kernel(nodes, item_history, itemrating_history, user_table, item_table,
           rating_table, ln1_w, ln1_b, ln2_w, ln2_b, ln3_w, ln3_b,
           att1_w, att1_b, att2_w, att2_b, att3_w, att3_b):
    B, L = item_history.shape
    D = user_table.shape[1]

    idx_f = item_history.astype(jnp.float32).T.reshape(-1)
    idx_items = idx_f.astype(jnp.int32)
    item_pairs = item_table.reshape(-1, 2 * D)
    gp3 = _sc_gather_items(item_pairs, idx_items >> 1, L, B, 256)

    parf = (item_history % 2).astype(jnp.float32).T.reshape(L, B, 1)
    ratf = itemrating_history.astype(jnp.float32).T.reshape(L, B, 1)
    nodes_b = nodes.astype(jnp.int32).reshape(B // _NB, 1, _NB)

    r1 = lambda b: b.reshape(1, -1)
    z = jnp.zeros((D, D), jnp.float32)
    weights = (
        jnp.concatenate([ln1_w[:, :D], z], axis=1),
        jnp.concatenate([z, ln1_w[:, :D]], axis=1),
        ln1_w[:, D:],
        r1(ln1_b),
        att1_w[:, :D],
        att1_w[:, D:],
        r1(att1_b),
        att2_w, r1(att2_b), att3_w,
        ln2_w, r1(ln2_b),
        ln3_w[:, :D],
        ln3_w[:, D:],
        r1(ln3_b),
    )
    return _tc_compute(gp3, parf, ratf, nodes_b, user_table, rating_table,
                       weights)


# speedup vs baseline: 1.2227x; 1.2227x over previous
import jax
import jax.numpy as jnp
from jax.experimental import pallas as pl
from jax.experimental.pallas import tpu as pltpu
from jax.experimental.pallas import tpu_sc as plsc


_NB = 128


def _sc_gather_items(table, idx, n_l, n_b, window):
    d = table.shape[1]
    n = idx.shape[0]
    per_l = n_b // window
    mesh = plsc.VectorSubcoreMesh(core_axis_name="core", subcore_axis_name="subcore")

    @pl.kernel(out_type=jax.ShapeDtypeStruct((n_l, n_b, d), table.dtype),
               mesh=mesh)
    def kern(tab_hbm, i_hbm, o_hbm):
        def body(i_vmem, o_vmem):
            pltpu.sync_copy(tab_hbm.at[i_vmem.at[0]], o_vmem.at[0])

        pltpu.emit_pipeline(
            body,
            grid=(n // window,),
            in_specs=[pl.BlockSpec((1, window), lambda i: (0, i))],
            out_specs=[pl.BlockSpec((1, window, d),
                                    lambda i: (i // per_l, i % per_l, 0))],
            core_axis_name=("core", "subcore"),
            dimension_semantics=(pltpu.PARALLEL,),
        )(i_hbm, o_hbm)

    return kern(table, idx.reshape(1, n))


def _tc_body(gp_ref, cond_ref, nodes_ref, utab_ref, rtab_ref,
             wiA_ref, wiB_ref, wr_ref, ln1b_ref, wa_ref, wau_ref, att1b_ref,
             att2w_ref, att2b_ref, att3w_ref, ln2w_ref, ln2b_ref,
             w3u_ref, w3i_ref, ln3b_ref, out_ref, u_scr, u_sem):
    f32 = jnp.float32

    def dot_t(x, w):
        return jax.lax.dot_general(x, w, (((1,), (1,)), ((), ())),
                                   preferred_element_type=f32)

    L, nB, D2 = gp_ref.shape
    D = D2 // 2

    def issue(j, _):
        r = nodes_ref[0, 0, j]
        pltpu.make_async_copy(utab_ref.at[pl.ds(r, 1), :],
                              u_scr.at[pl.ds(j, 1), :], u_sem).start()
        return _
    jax.lax.fori_loop(0, nB, issue, None)

    gp2 = gp_ref[...].reshape(L * nB, D2)
    even_part = dot_t(gp2, wiA_ref[...]).reshape(L, nB, D)
    odd_part = dot_t(gp2, wiB_ref[...]).reshape(L, nB, D)
    c2 = cond_ref[...]
    cb = jax.lax.broadcast_in_dim(c2, (L, nB, D), (0, 1))
    par3 = cb >= 8.0
    r3 = jnp.where(par3, cb - 16.0, cb)
    item_part = jnp.where(par3, odd_part, even_part)

    pre_rat = dot_t(rtab_ref[...], wr_ref[...]) + ln1b_ref[...]
    p = [pre_rat[k:k + 1][None] for k in range(5)]
    v01 = jnp.where(r3 < 0.5, p[0], p[1])
    v23 = jnp.where(r3 < 2.5, p[2], p[3])
    v0123 = jnp.where(r3 < 1.5, v01, v23)
    ratc = jnp.where(r3 < 3.5, v0123, p[4])
    xr3 = jnp.maximum(item_part + ratc, 0.0)
    xr2 = xr3.reshape(L * nB, D)

    def drain(j, _):
        pltpu.make_async_copy(utab_ref.at[pl.ds(0, 1), :],
                              u_scr.at[pl.ds(0, 1), :], u_sem).wait()
        return _
    jax.lax.fori_loop(0, nB, drain, None)
    u = u_scr[...]

    u_att = dot_t(u, wau_ref[...]) + att1b_ref[...]
    a1 = dot_t(xr2, wa_ref[...]).reshape(L, nB, D) + u_att[None]
    a1 = jnp.maximum(a1, 0.0)
    a2 = jnp.maximum(dot_t(a1.reshape(L * nB, D), att2w_ref[...])
                     + att2b_ref[...], 0.0)

    s = jnp.sum(a2.reshape(L, nB, D) * att3w_ref[...][None], axis=2,
                keepdims=True)
    m = jnp.max(s, axis=0, keepdims=True)
    e = jnp.exp(s - m)
    denom = jnp.sum(e, axis=0, keepdims=True)
    hI = jnp.sum(xr3 * (e / denom), axis=0)

    h2 = jnp.maximum(dot_t(hI, ln2w_ref[...]) + ln2b_ref[...], 0.0)
    out = dot_t(u, w3u_ref[...]) + dot_t(h2, w3i_ref[...]) + ln3b_ref[...]
    out_ref[...] = jnp.maximum(out, 0.0)


def _tc_compute(gp3, cond, nodes_b, user_table, rating_table, weights):
    L, B, D2 = gp3.shape
    D = D2 // 2
    grid = (B // _NB,)
    full = lambda a: pl.BlockSpec(a.shape, lambda i: tuple(0 for _ in a.shape))
    return pl.pallas_call(
        _tc_body,
        grid=grid,
        in_specs=[
            pl.BlockSpec((L, _NB, D2), lambda i: (0, i, 0)),
            pl.BlockSpec((L, _NB), lambda i: (0, i)),
            pl.BlockSpec((1, 1, _NB), lambda i: (i, 0, 0),
                         memory_space=pltpu.SMEM),
            pl.BlockSpec(memory_space=pl.ANY),
            full(rating_table),
        ] + [full(w) for w in weights],
        out_specs=pl.BlockSpec((_NB, D), lambda i: (i, 0)),
        out_shape=jax.ShapeDtypeStruct((B, D), jnp.float32),
        scratch_shapes=[pltpu.VMEM((_NB, D), jnp.float32),
                        pltpu.SemaphoreType.DMA],
        compiler_params=pltpu.CompilerParams(
            dimension_semantics=("parallel",)),
    )(gp3, cond, nodes_b, user_table, rating_table, *weights)


def kernel(nodes, item_history, itemrating_history, user_table, item_table,
           rating_table, ln1_w, ln1_b, ln2_w, ln2_b, ln3_w, ln3_b,
           att1_w, att1_b, att2_w, att2_b, att3_w, att3_b):
    B, L = item_history.shape
    D = user_table.shape[1]

    idx_f = item_history.astype(jnp.float32).T.reshape(-1)
    idx_items = idx_f.astype(jnp.int32)
    item_pairs = item_table.reshape(-1, 2 * D)
    gp3 = _sc_gather_items(item_pairs, idx_items >> 1, L, B, 256)

    cond = (itemrating_history.astype(jnp.float32)
            + 16.0 * (item_history % 2).astype(jnp.float32)).T
    nodes_b = nodes.astype(jnp.int32).reshape(B // _NB, 1, _NB)

    r1 = lambda b: b.reshape(1, -1)
    z = jnp.zeros((D, D), jnp.float32)
    weights = (
        jnp.concatenate([ln1_w[:, :D], z], axis=1),
        jnp.concatenate([z, ln1_w[:, :D]], axis=1),
        ln1_w[:, D:],
        r1(ln1_b),
        att1_w[:, :D],
        att1_w[:, D:],
        r1(att1_b),
        att2_w, r1(att2_b), att3_w,
        ln2_w, r1(ln2_b),
        ln3_w[:, :D],
        ln3_w[:, D:],
        r1(ln3_b),
    )
    return _tc_compute(gp3, cond, nodes_b, user_table, rating_table, weights)
